# baseline (speedup 1.0000x reference)
import jax
import jax.numpy as jnp
from jax import lax
from jax.experimental import pallas as pl
from jax.experimental.pallas import tpu as pltpu

_BATCH = 16384
_DIM = 64
_BLK = 512
_NBLK = _BATCH // _BLK
_RTB = 1000
_RTB_PAD = 1024
_G8 = 8


def _body(aidx_ref, ent_ref, rel_ref, rtb_ref, w1t_ref, w2t_ref, b_ref,
          o_ref, buf, m2_s, sems):
    i = pl.program_id(0)

    def fire_eight(j, s, k8):
        k0 = k8 * _G8
        idxs = [aidx_ref[j * _BLK + k0 + u] for u in range(_G8)]
        for u in range(_G8):
            pltpu.make_async_copy(
                ent_ref.at[pl.ds(idxs[u], 1)],
                buf.at[s, pl.ds(k0 + u, 1)],
                sems.at[s, u % 2],
            ).start(priority=u % 2)

    def fire_block(j, s):
        def go(k8, carry):
            fire_eight(j, s, k8)
            return carry

        lax.fori_loop(0, _BLK // _G8, go, 0)

    def wait_block(s):
        for t in range(2):
            pltpu.make_async_copy(
                ent_ref.at[pl.ds(0, _BLK // 2)],
                buf.at[s, pl.ds(0, _BLK // 2)],
                sems.at[s, t],
            ).wait()

    @pl.when(i == 0)
    def _():
        m2_s[...] = jnp.dot(
            rtb_ref[...], w2t_ref[...], preferred_element_type=jnp.float32
        )
        fire_block(0, 0)

    wait_block(i % 2)

    lanes8 = lax.broadcasted_iota(jnp.int32, (_G8, _RTB_PAD), 1)

    def group(k8, carry):
        @pl.when(i + 1 < _NBLK)
        def _():
            fire_eight(i + 1, (i + 1) % 2, k8)

        k0 = k8 * _G8
        a8 = buf[i % 2, pl.ds(k0, _G8)]
        rid8 = rel_ref[pl.ds(k0, _G8)]
        onehot8 = jnp.where(lanes8 == rid8, 1.0, 0.0).astype(jnp.float32)
        acc = jnp.dot(a8, w1t_ref[...], preferred_element_type=jnp.float32)
        acc += jnp.dot(onehot8, m2_s[...], preferred_element_type=jnp.float32)
        o_ref[pl.ds(k0, _G8)] = jnp.maximum(acc + b_ref[...], 0.0)
        return carry

    lax.fori_loop(0, _BLK // _G8, group, 0)


@jax.jit
def _run(entity_table, relation_table, W, b, anchor, rel):
    wt = W.T
    w1t = wt[:_DIM]
    w2t = wt[_DIM:]
    b2d = b.reshape(1, _DIM)
    r2d = rel.reshape(_BATCH, 1)
    rtb_pad = jnp.pad(relation_table, ((0, _RTB_PAD - _RTB), (0, 0)))
    grid_spec = pltpu.PrefetchScalarGridSpec(
        num_scalar_prefetch=1,
        grid=(_NBLK,),
        in_specs=[
            pl.BlockSpec(memory_space=pltpu.HBM),
            pl.BlockSpec((_BLK, 1), lambda i, aref: (i, 0)),
            pl.BlockSpec((_RTB_PAD, _DIM), lambda i, aref: (0, 0)),
            pl.BlockSpec((_DIM, _DIM), lambda i, aref: (0, 0)),
            pl.BlockSpec((_DIM, _DIM), lambda i, aref: (0, 0)),
            pl.BlockSpec((1, _DIM), lambda i, aref: (0, 0)),
        ],
        out_specs=pl.BlockSpec((_BLK, _DIM), lambda i, aref: (i, 0)),
        scratch_shapes=[
            pltpu.VMEM((2, _BLK, _DIM), jnp.float32),
            pltpu.VMEM((_RTB_PAD, _DIM), jnp.float32),
            pltpu.SemaphoreType.DMA((2, 2)),
        ],
    )
    out = pl.pallas_call(
        _body,
        grid_spec=grid_spec,
        out_shape=jax.ShapeDtypeStruct((_BATCH, _DIM), jnp.float32),
        compiler_params=pltpu.CompilerParams(
            dimension_semantics=("arbitrary",),
        ),
    )(anchor, entity_table, r2d, rtb_pad, w1t, w2t, b2d)
    return out


def kernel(entity_table, relation_table, W, b, anchor, rel):
    return _run(entity_table, relation_table, W, b, anchor, rel)


# speedup vs baseline: 1.9323x; 1.9323x over previous
import jax
import jax.numpy as jnp
from jax import lax
from jax.experimental import pallas as pl
from jax.experimental.pallas import tpu as pltpu

_BATCH = 16384
_DIM = 64
_BLK = 512
_NBLK = _BATCH // _BLK
_RTB = 1000
_RTB_PAD = 1024


def _body(aidx_ref, ent_ref, rel_ref, rtb_ref, w1t_ref, w2t_ref, b_ref,
          o_ref, buf, m2_s, sems):
    i = pl.program_id(0)

    def fire_block(j, s):
        def fire_eight(k8, carry):
            k0 = k8 * 8
            idxs = [aidx_ref[j * _BLK + k0 + u] for u in range(8)]
            for u in range(8):
                pltpu.make_async_copy(
                    ent_ref.at[pl.ds(idxs[u], 1)],
                    buf.at[s, pl.ds(k0 + u, 1)],
                    sems.at[s],
                ).start(priority=u % 2)
            return carry

        lax.fori_loop(0, _BLK // 8, fire_eight, 0)

    def wait_block(s):
        pltpu.make_async_copy(
            ent_ref.at[pl.ds(0, _BLK)],
            buf.at[s],
            sems.at[s],
        ).wait()

    @pl.when(i == 0)
    def _():
        m2_s[...] = jnp.dot(
            rtb_ref[...], w2t_ref[...], preferred_element_type=jnp.float32
        )
        fire_block(0, 0)
        fire_block(1, 1)
        fire_block(2, 2)

    @pl.when(i + 3 < _NBLK)
    def _():
        fire_block(i + 3, (i + 3) % 4)

    wait_block(i % 4)

    a = buf[i % 4]
    rid = rel_ref[...]
    lanes = lax.broadcasted_iota(jnp.int32, (_BLK, _RTB_PAD), 1)
    onehot = jnp.where(lanes == rid, 1.0, 0.0).astype(jnp.float32)
    acc = jnp.dot(a, w1t_ref[...], preferred_element_type=jnp.float32)
    acc += jnp.dot(onehot, m2_s[...], preferred_element_type=jnp.float32)
    o_ref[...] = jnp.maximum(acc + b_ref[...], 0.0)


@jax.jit
def _run(entity_table, relation_table, W, b, anchor, rel):
    wt = W.T
    w1t = wt[:_DIM]
    w2t = wt[_DIM:]
    b2d = b.reshape(1, _DIM)
    r2d = rel.reshape(_BATCH, 1)
    rtb_pad = jnp.pad(relation_table, ((0, _RTB_PAD - _RTB), (0, 0)))
    grid_spec = pltpu.PrefetchScalarGridSpec(
        num_scalar_prefetch=1,
        grid=(_NBLK,),
        in_specs=[
            pl.BlockSpec(memory_space=pltpu.HBM),
            pl.BlockSpec((_BLK, 1), lambda i, aref: (i, 0)),
            pl.BlockSpec((_RTB_PAD, _DIM), lambda i, aref: (0, 0)),
            pl.BlockSpec((_DIM, _DIM), lambda i, aref: (0, 0)),
            pl.BlockSpec((_DIM, _DIM), lambda i, aref: (0, 0)),
            pl.BlockSpec((1, _DIM), lambda i, aref: (0, 0)),
        ],
        out_specs=pl.BlockSpec((_BLK, _DIM), lambda i, aref: (i, 0)),
        scratch_shapes=[
            pltpu.VMEM((4, _BLK, _DIM), jnp.float32),
            pltpu.VMEM((_RTB_PAD, _DIM), jnp.float32),
            pltpu.SemaphoreType.DMA((4,)),
        ],
    )
    out = pl.pallas_call(
        _body,
        grid_spec=grid_spec,
        out_shape=jax.ShapeDtypeStruct((_BATCH, _DIM), jnp.float32),
        compiler_params=pltpu.CompilerParams(
            dimension_semantics=("arbitrary",),
        ),
    )(anchor, entity_table, r2d, rtb_pad, w1t, w2t, b2d)
    return out


def kernel(entity_table, relation_table, W, b, anchor, rel):
    return _run(entity_table, relation_table, W, b, anchor, rel)


# speedup vs baseline: 1.9362x; 1.0020x over previous
import jax
import jax.numpy as jnp
from jax import lax
from jax.experimental import pallas as pl
from jax.experimental.pallas import tpu as pltpu

_BATCH = 16384
_DIM = 64
_BLK = 512
_NBLK = _BATCH // _BLK
_RTB = 1000
_RTB_PAD = 1024


def _body(aidx_ref, ent_ref, rel_ref, rtb_ref, w1t_ref, w2t_ref, b_ref,
          o_ref, buf, m2_s, sems):
    i = pl.program_id(0)

    def fire_block(j, s):
        def fire_eight(k8, carry):
            k0 = k8 * 8
            idxs = [aidx_ref[j * _BLK + k0 + u] for u in range(8)]
            for u in range(8):
                pltpu.make_async_copy(
                    ent_ref.at[pl.ds(idxs[u], 1)],
                    buf.at[s, pl.ds(k0 + u, 1)],
                    sems.at[s, u % 2],
                ).start(priority=u % 2)
            return carry

        lax.fori_loop(0, _BLK // 8, fire_eight, 0)

    def wait_block(s):
        for t in range(2):
            pltpu.make_async_copy(
                ent_ref.at[pl.ds(0, _BLK // 2)],
                buf.at[s, pl.ds(0, _BLK // 2)],
                sems.at[s, t],
            ).wait()

    @pl.when(i == 0)
    def _():
        m2_s[...] = jnp.dot(
            rtb_ref[...], w2t_ref[...], preferred_element_type=jnp.float32
        )
        fire_block(0, 0)
        fire_block(1, 1)
        fire_block(2, 2)

    @pl.when(i + 3 < _NBLK)
    def _():
        fire_block(i + 3, (i + 3) % 4)

    wait_block(i % 4)

    a = buf[i % 4]
    rid = rel_ref[...]
    lanes = lax.broadcasted_iota(jnp.int32, (_BLK, _RTB_PAD), 1)
    onehot = jnp.where(lanes == rid, 1.0, 0.0).astype(jnp.float32)
    acc = jnp.dot(a, w1t_ref[...], preferred_element_type=jnp.float32)
    acc += jnp.dot(onehot, m2_s[...], preferred_element_type=jnp.float32)
    o_ref[...] = jnp.maximum(acc + b_ref[...], 0.0)


@jax.jit
def _run(entity_table, relation_table, W, b, anchor, rel):
    wt = W.T
    w1t = wt[:_DIM]
    w2t = wt[_DIM:]
    b2d = b.reshape(1, _DIM)
    r2d = rel.reshape(_BATCH, 1)
    rtb_pad = jnp.pad(relation_table, ((0, _RTB_PAD - _RTB), (0, 0)))
    grid_spec = pltpu.PrefetchScalarGridSpec(
        num_scalar_prefetch=1,
        grid=(_NBLK,),
        in_specs=[
            pl.BlockSpec(memory_space=pltpu.HBM),
            pl.BlockSpec((_BLK, 1), lambda i, aref: (i, 0)),
            pl.BlockSpec((_RTB_PAD, _DIM), lambda i, aref: (0, 0)),
            pl.BlockSpec((_DIM, _DIM), lambda i, aref: (0, 0)),
            pl.BlockSpec((_DIM, _DIM), lambda i, aref: (0, 0)),
            pl.BlockSpec((1, _DIM), lambda i, aref: (0, 0)),
        ],
        out_specs=pl.BlockSpec((_BLK, _DIM), lambda i, aref: (i, 0)),
        scratch_shapes=[
            pltpu.VMEM((4, _BLK, _DIM), jnp.float32),
            pltpu.VMEM((_RTB_PAD, _DIM), jnp.float32),
            pltpu.SemaphoreType.DMA((4, 2)),
        ],
    )
    out = pl.pallas_call(
        _body,
        grid_spec=grid_spec,
        out_shape=jax.ShapeDtypeStruct((_BATCH, _DIM), jnp.float32),
        compiler_params=pltpu.CompilerParams(
            dimension_semantics=("arbitrary",),
        ),
    )(anchor, entity_table, r2d, rtb_pad, w1t, w2t, b2d)
    return out


def kernel(entity_table, relation_table, W, b, anchor, rel):
    return _run(entity_table, relation_table, W, b, anchor, rel)


# speedup vs baseline: 1.9560x; 1.0102x over previous
import jax
import jax.numpy as jnp
from jax import lax
from jax.experimental import pallas as pl
from jax.experimental.pallas import tpu as pltpu

_BATCH = 16384
_DIM = 64
_BLK = 1024
_NBLK = _BATCH // _BLK
_RTB = 1000
_RTB_PAD = 1024


def _body(aidx_ref, ent_ref, rel_ref, rtb_ref, w1t_ref, w2t_ref, b_ref,
          o_ref, buf, m2_s, sems):
    i = pl.program_id(0)

    def fire_block(j, s):
        def fire_eight(k8, carry):
            k0 = k8 * 8
            idxs = [aidx_ref[j * _BLK + k0 + u] for u in range(8)]
            for u in range(8):
                pltpu.make_async_copy(
                    ent_ref.at[pl.ds(idxs[u], 1)],
                    buf.at[s, pl.ds(k0 + u, 1)],
                    sems.at[s],
                ).start(priority=u % 2)
            return carry

        lax.fori_loop(0, _BLK // 8, fire_eight, 0)

    def wait_block(s):
        pltpu.make_async_copy(
            ent_ref.at[pl.ds(0, _BLK)],
            buf.at[s],
            sems.at[s],
        ).wait()

    @pl.when(i == 0)
    def _():
        m2_s[...] = jnp.dot(
            rtb_ref[...], w2t_ref[...], preferred_element_type=jnp.float32
        )
        fire_block(0, 0)
        fire_block(1, 1)
        fire_block(2, 2)

    @pl.when(i + 3 < _NBLK)
    def _():
        fire_block(i + 3, (i + 3) % 4)

    wait_block(i % 4)

    a = buf[i % 4]
    rid = rel_ref[...]
    lanes = lax.broadcasted_iota(jnp.int32, (_BLK, _RTB_PAD), 1)
    onehot = jnp.where(lanes == rid, 1.0, 0.0).astype(jnp.float32)
    acc = jnp.dot(a, w1t_ref[...], preferred_element_type=jnp.float32)
    acc += jnp.dot(onehot, m2_s[...], preferred_element_type=jnp.float32)
    o_ref[...] = jnp.maximum(acc + b_ref[...], 0.0)


@jax.jit
def _run(entity_table, relation_table, W, b, anchor, rel):
    wt = W.T
    w1t = wt[:_DIM]
    w2t = wt[_DIM:]
    b2d = b.reshape(1, _DIM)
    r2d = rel.reshape(_BATCH, 1)
    rtb_pad = jnp.pad(relation_table, ((0, _RTB_PAD - _RTB), (0, 0)))
    grid_spec = pltpu.PrefetchScalarGridSpec(
        num_scalar_prefetch=1,
        grid=(_NBLK,),
        in_specs=[
            pl.BlockSpec(memory_space=pltpu.HBM),
            pl.BlockSpec((_BLK, 1), lambda i, aref: (i, 0)),
            pl.BlockSpec((_RTB_PAD, _DIM), lambda i, aref: (0, 0)),
            pl.BlockSpec((_DIM, _DIM), lambda i, aref: (0, 0)),
            pl.BlockSpec((_DIM, _DIM), lambda i, aref: (0, 0)),
            pl.BlockSpec((1, _DIM), lambda i, aref: (0, 0)),
        ],
        out_specs=pl.BlockSpec((_BLK, _DIM), lambda i, aref: (i, 0)),
        scratch_shapes=[
            pltpu.VMEM((4, _BLK, _DIM), jnp.float32),
            pltpu.VMEM((_RTB_PAD, _DIM), jnp.float32),
            pltpu.SemaphoreType.DMA((4,)),
        ],
    )
    out = pl.pallas_call(
        _body,
        grid_spec=grid_spec,
        out_shape=jax.ShapeDtypeStruct((_BATCH, _DIM), jnp.float32),
        compiler_params=pltpu.CompilerParams(
            dimension_semantics=("arbitrary",),
        ),
    )(anchor, entity_table, r2d, rtb_pad, w1t, w2t, b2d)
    return out


def kernel(entity_table, relation_table, W, b, anchor, rel):
    return _run(entity_table, relation_table, W, b, anchor, rel)


# speedup vs baseline: 1.9626x; 1.0033x over previous
import jax
import jax.numpy as jnp
from jax import lax
from jax.experimental import pallas as pl
from jax.experimental.pallas import tpu as pltpu

_BATCH = 16384
_DIM = 64
_BLK = 2048
_NBLK = _BATCH // _BLK
_RTB = 1000
_RTB_PAD = 1024


def _body(aidx_ref, ent_ref, rel_ref, rtb_ref, w1t_ref, w2t_ref, b_ref,
          o_ref, buf, m2_s, sems):
    i = pl.program_id(0)

    def fire_block(j, s):
        def fire_eight(k8, carry):
            k0 = k8 * 8
            idxs = [aidx_ref[j * _BLK + k0 + u] for u in range(8)]
            for u in range(8):
                pltpu.make_async_copy(
                    ent_ref.at[pl.ds(idxs[u], 1)],
                    buf.at[s, pl.ds(k0 + u, 1)],
                    sems.at[s],
                ).start(priority=u % 2)
            return carry

        lax.fori_loop(0, _BLK // 8, fire_eight, 0)

    def wait_block(s):
        pltpu.make_async_copy(
            ent_ref.at[pl.ds(0, _BLK)],
            buf.at[s],
            sems.at[s],
        ).wait()

    @pl.when(i == 0)
    def _():
        m2_s[...] = jnp.dot(
            rtb_ref[...], w2t_ref[...], preferred_element_type=jnp.float32
        )
        fire_block(0, 0)
        fire_block(1, 1)
        fire_block(2, 2)

    @pl.when(i + 3 < _NBLK)
    def _():
        fire_block(i + 3, (i + 3) % 4)

    wait_block(i % 4)

    a = buf[i % 4]
    rid = rel_ref[...]
    lanes = lax.broadcasted_iota(jnp.int32, (_BLK, _RTB_PAD), 1)
    onehot = jnp.where(lanes == rid, 1.0, 0.0).astype(jnp.float32)
    acc = jnp.dot(a, w1t_ref[...], preferred_element_type=jnp.float32)
    acc += jnp.dot(onehot, m2_s[...], preferred_element_type=jnp.float32)
    o_ref[...] = jnp.maximum(acc + b_ref[...], 0.0)


@jax.jit
def _run(entity_table, relation_table, W, b, anchor, rel):
    wt = W.T
    w1t = wt[:_DIM]
    w2t = wt[_DIM:]
    b2d = b.reshape(1, _DIM)
    r2d = rel.reshape(_BATCH, 1)
    rtb_pad = jnp.pad(relation_table, ((0, _RTB_PAD - _RTB), (0, 0)))
    grid_spec = pltpu.PrefetchScalarGridSpec(
        num_scalar_prefetch=1,
        grid=(_NBLK,),
        in_specs=[
            pl.BlockSpec(memory_space=pltpu.HBM),
            pl.BlockSpec((_BLK, 1), lambda i, aref: (i, 0)),
            pl.BlockSpec((_RTB_PAD, _DIM), lambda i, aref: (0, 0)),
            pl.BlockSpec((_DIM, _DIM), lambda i, aref: (0, 0)),
            pl.BlockSpec((_DIM, _DIM), lambda i, aref: (0, 0)),
            pl.BlockSpec((1, _DIM), lambda i, aref: (0, 0)),
        ],
        out_specs=pl.BlockSpec((_BLK, _DIM), lambda i, aref: (i, 0)),
        scratch_shapes=[
            pltpu.VMEM((4, _BLK, _DIM), jnp.float32),
            pltpu.VMEM((_RTB_PAD, _DIM), jnp.float32),
            pltpu.SemaphoreType.DMA((4,)),
        ],
    )
    out = pl.pallas_call(
        _body,
        grid_spec=grid_spec,
        out_shape=jax.ShapeDtypeStruct((_BATCH, _DIM), jnp.float32),
        compiler_params=pltpu.CompilerParams(
            dimension_semantics=("arbitrary",),
        ),
    )(anchor, entity_table, r2d, rtb_pad, w1t, w2t, b2d)
    return out


def kernel(entity_table, relation_table, W, b, anchor, rel):
    return _run(entity_table, relation_table, W, b, anchor, rel)


# speedup vs baseline: 1.9772x; 1.0074x over previous
import jax
import jax.numpy as jnp
from jax import lax
from jax.experimental import pallas as pl
from jax.experimental.pallas import tpu as pltpu

_BATCH = 16384
_DIM = 64
_BLK = 4096
_NBLK = _BATCH // _BLK
_RTB = 1000
_RTB_PAD = 1024


def _body(aidx_ref, ent_ref, rel_ref, rtb_ref, w1t_ref, w2t_ref, b_ref,
          o_ref, buf, m2_s, sems):
    i = pl.program_id(0)

    def fire_block(j, s):
        def fire_eight(k8, carry):
            k0 = k8 * 8
            idxs = [aidx_ref[j * _BLK + k0 + u] for u in range(8)]
            for u in range(8):
                pltpu.make_async_copy(
                    ent_ref.at[pl.ds(idxs[u], 1)],
                    buf.at[s, pl.ds(k0 + u, 1)],
                    sems.at[s],
                ).start(priority=u % 2)
            return carry

        lax.fori_loop(0, _BLK // 8, fire_eight, 0)

    def wait_block(s):
        pltpu.make_async_copy(
            ent_ref.at[pl.ds(0, _BLK)],
            buf.at[s],
            sems.at[s],
        ).wait()

    @pl.when(i == 0)
    def _():
        m2_s[...] = jnp.dot(
            rtb_ref[...], w2t_ref[...], preferred_element_type=jnp.float32
        )
        fire_block(0, 0)
        fire_block(1, 1)
        fire_block(2, 2)

    @pl.when(i + 3 < _NBLK)
    def _():
        fire_block(i + 3, (i + 3) % 4)

    wait_block(i % 4)

    a = buf[i % 4]
    rid = rel_ref[...]
    lanes = lax.broadcasted_iota(jnp.int32, (_BLK, _RTB_PAD), 1)
    onehot = jnp.where(lanes == rid, 1.0, 0.0).astype(jnp.float32)
    acc = jnp.dot(a, w1t_ref[...], preferred_element_type=jnp.float32)
    acc += jnp.dot(onehot, m2_s[...], preferred_element_type=jnp.float32)
    o_ref[...] = jnp.maximum(acc + b_ref[...], 0.0)


@jax.jit
def _run(entity_table, relation_table, W, b, anchor, rel):
    wt = W.T
    w1t = wt[:_DIM]
    w2t = wt[_DIM:]
    b2d = b.reshape(1, _DIM)
    r2d = rel.reshape(_BATCH, 1)
    rtb_pad = jnp.pad(relation_table, ((0, _RTB_PAD - _RTB), (0, 0)))
    grid_spec = pltpu.PrefetchScalarGridSpec(
        num_scalar_prefetch=1,
        grid=(_NBLK,),
        in_specs=[
            pl.BlockSpec(memory_space=pltpu.HBM),
            pl.BlockSpec((_BLK, 1), lambda i, aref: (i, 0)),
            pl.BlockSpec((_RTB_PAD, _DIM), lambda i, aref: (0, 0)),
            pl.BlockSpec((_DIM, _DIM), lambda i, aref: (0, 0)),
            pl.BlockSpec((_DIM, _DIM), lambda i, aref: (0, 0)),
            pl.BlockSpec((1, _DIM), lambda i, aref: (0, 0)),
        ],
        out_specs=pl.BlockSpec((_BLK, _DIM), lambda i, aref: (i, 0)),
        scratch_shapes=[
            pltpu.VMEM((4, _BLK, _DIM), jnp.float32),
            pltpu.VMEM((_RTB_PAD, _DIM), jnp.float32),
            pltpu.SemaphoreType.DMA((4,)),
        ],
    )
    out = pl.pallas_call(
        _body,
        grid_spec=grid_spec,
        out_shape=jax.ShapeDtypeStruct((_BATCH, _DIM), jnp.float32),
        compiler_params=pltpu.CompilerParams(
            dimension_semantics=("arbitrary",),
        ),
    )(anchor, entity_table, r2d, rtb_pad, w1t, w2t, b2d)
    return out


def kernel(entity_table, relation_table, W, b, anchor, rel):
    return _run(entity_table, relation_table, W, b, anchor, rel)
